# baseline (speedup 1.0000x reference)
import jax
import jax.numpy as jnp
from jax import lax
from jax.experimental import pallas as pl
from jax.experimental.pallas import tpu as pltpu

_PROMPT_K = 16
_TOPK = 2
_H_BLK = 512


def _gelu(v):
    return 0.5 * v * (1.0 + lax.erf(v * 0.7071067811865476))


def _router_kernel(x_ref, w1_ref, b1_ref, w2_ref, b2_ref,
                   probs_ref, stats_ref, emask_ref, twf_ref, wvec_ref,
                   idx_ref, xbf_ref, xst_ref):
    i = pl.program_id(0)
    nt = pl.num_programs(0)
    blk = x_ref.shape[0]
    xt = x_ref[...]
    xbf_ref[...] = xt.astype(jnp.bfloat16)
    xst_ref[pl.ds(i * blk, blk), :] = xt

    @pl.when(i == nt - 1)
    def _router_tail():
        _router_compute(xst_ref, w1_ref, b1_ref, w2_ref, b2_ref, probs_ref,
                        stats_ref, emask_ref, twf_ref, wvec_ref, idx_ref)


def _router_compute(x_ref, w1_ref, b1_ref, w2_ref, b2_ref,
                    probs_ref, stats_ref, emask_ref, twf_ref, wvec_ref,
                    idx_ref):
    n, c = x_ref.shape
    e = w2_ref.shape[0]
    x = x_ref[...]
    xm = jnp.sum(x, axis=0, keepdims=True) * (1.0 / n)
    num = jnp.sum(x * xm, axis=1, keepdims=True)
    xn = jnp.sqrt(jnp.sum(x * x, axis=1, keepdims=True))
    mn = jnp.sqrt(jnp.sum(xm * xm))
    eps = 1e-8
    sim = num / (jnp.maximum(xn, eps) * jnp.maximum(mn, eps))

    iota_n = lax.broadcasted_iota(jnp.int32, (n, 1), 0)

    def body(_, carry):
        simw, mask = carry
        m = jnp.max(simw)
        idx = jnp.min(jnp.where(simw == m, iota_n, n))
        hit = iota_n == idx
        mask = jnp.where(hit, 1.0, mask)
        simw = jnp.where(hit, -jnp.inf, simw)
        return simw, mask

    _, mask = lax.fori_loop(0, _PROMPT_K, body,
                            (sim, jnp.zeros((n, 1), jnp.float32)))
    context = jnp.sum(x * mask, axis=0, keepdims=True) * (1.0 / _PROMPT_K)

    h = _gelu(lax.dot_general(context, w1_ref[...],
                              (((1,), (1,)), ((), ())),
                              preferred_element_type=jnp.float32)
              + b1_ref[...])
    logits = lax.dot_general(h, w2_ref[...],
                             (((1,), (1,)), ((), ())),
                             preferred_element_type=jnp.float32) + b2_ref[...]
    ex = jnp.exp(logits - jnp.max(logits))
    p = ex / jnp.sum(ex)

    iota_e = lax.broadcasted_iota(jnp.int32, (1, e), 1)
    m1 = jnp.max(p)
    i1 = jnp.min(jnp.where(p == m1, iota_e, e))
    p2 = jnp.where(iota_e == i1, -1.0, p)
    m2 = jnp.max(p2)
    i2 = jnp.min(jnp.where(p2 == m2, iota_e, e))
    s = m1 + m2 + 1e-9
    tw0 = m1 / s
    tw1 = m2 / s

    local_ent = jnp.sum(-p * jnp.log(p + 1e-6))
    global_ent = jnp.sum(p * jnp.log(p + 1e-6))

    w2m = w2_ref[...]
    rn = jnp.sqrt(jnp.sum(w2m * w2m, axis=1, keepdims=True))
    wn = w2m / jnp.maximum(rn, 1e-12)
    simm = lax.dot_general(wn, wn, (((1,), (1,)), ((), ())),
                           preferred_element_type=jnp.float32)
    eye = (lax.broadcasted_iota(jnp.int32, (e, e), 0)
           == lax.broadcasted_iota(jnp.int32, (e, e), 1)).astype(jnp.float32)
    ortho = jnp.sqrt(jnp.sum((simm - eye) ** 2))

    hot1 = iota_e == i1
    hot2 = iota_e == i2
    probs_ref[...] = p
    emask_ref[...] = (hot1 | hot2).astype(jnp.float32)
    twf_ref[...] = jnp.where(hot1, tw0, 0.0) + jnp.where(hot2, tw1, 0.0)
    stats_ref[...] = (jnp.where(iota_e == 0, local_ent, 0.0)
                      + jnp.where(iota_e == 1, global_ent, 0.0)
                      + jnp.where(iota_e == 2, ortho, 0.0))
    wvec_ref[...] = (jnp.where(iota_e == 0, tw0, 0.0)
                     + jnp.where(iota_e == 1, tw1, 0.0))
    idx_ref[...] = (jnp.where(iota_e == 0, i1, 0)
                    + jnp.where(iota_e == 1, i2, 0)).astype(jnp.int32)


def _expert_kernel(idx_ref, w_ref, x_ref,
                   we1a_ref, we1b_ref, be1a_ref, be1b_ref,
                   we2a_ref, we2b_ref, be2a_ref, be2b_ref, out_ref):
    h = pl.program_id(0)
    lane = lax.broadcasted_iota(jnp.int32, w_ref.shape, 1)
    w0 = jnp.sum(jnp.where(lane == 0, w_ref[...], 0.0))
    w1 = jnp.sum(jnp.where(lane == 1, w_ref[...], 0.0))

    x = x_ref[...]
    bf = jnp.bfloat16
    f32 = jnp.float32
    hpa = jnp.dot(x, we1a_ref[0].astype(bf), preferred_element_type=f32)
    ga = _gelu((hpa + be1a_ref[0]).astype(bf))
    hpb = jnp.dot(x, we1b_ref[0].astype(bf), preferred_element_type=f32)
    gb = _gelu((hpb + be1b_ref[0]).astype(bf))
    o = (w0 * jnp.dot(ga, we2a_ref[0].astype(bf), preferred_element_type=f32)
         + w1 * jnp.dot(gb, we2b_ref[0].astype(bf),
                        preferred_element_type=f32))

    @pl.when(h == 0)
    def _():
        out_ref[...] = (w0 * be2a_ref[0] + w1 * be2b_ref[0]) + o

    @pl.when(h != 0)
    def _():
        out_ref[...] += o


def kernel(x, W1, b1, W2, b2, We1, be1, We2, be2):
    b, n, c = x.shape
    e, _, hid = We1.shape
    xs = x.reshape(n, c)

    r_blk = 256
    r_nt = n // r_blk
    small = lambda i: (0, 0)
    probs, stats, emask, twf, wvec, idx8, xbf = pl.pallas_call(
        _router_kernel,
        grid=(r_nt,),
        in_specs=[
            pl.BlockSpec((r_blk, c), lambda i: (i, 0)),
            pl.BlockSpec(W1.shape, small),
            pl.BlockSpec((1, b1.shape[0]), small),
            pl.BlockSpec(W2.shape, small),
            pl.BlockSpec((1, b2.shape[0]), small),
        ],
        out_specs=(
            pl.BlockSpec((1, e), small),
            pl.BlockSpec((1, e), small),
            pl.BlockSpec((1, e), small),
            pl.BlockSpec((1, e), small),
            pl.BlockSpec((1, e), small),
            pl.BlockSpec((1, e), small),
            pl.BlockSpec((r_blk, c), lambda i: (i, 0)),
        ),
        out_shape=(
            jax.ShapeDtypeStruct((1, e), jnp.float32),
            jax.ShapeDtypeStruct((1, e), jnp.float32),
            jax.ShapeDtypeStruct((1, e), jnp.float32),
            jax.ShapeDtypeStruct((1, e), jnp.float32),
            jax.ShapeDtypeStruct((1, e), jnp.float32),
            jax.ShapeDtypeStruct((1, e), jnp.int32),
            jax.ShapeDtypeStruct((n, c), jnp.bfloat16),
        ),
        scratch_shapes=[pltpu.VMEM((n, c), jnp.float32)],
    )(xs, W1, b1.reshape(1, -1), W2, b2.reshape(1, -1))

    topk_i = idx8[:, :_TOPK]

    ht = hid // _H_BLK
    grid_spec = pltpu.PrefetchScalarGridSpec(
        num_scalar_prefetch=1,
        grid=(ht,),
        in_specs=[
            pl.BlockSpec((1, 8), lambda h, idx: (0, 0)),
            pl.BlockSpec((n, c), lambda h, idx: (0, 0)),
            pl.BlockSpec((1, c, _H_BLK), lambda h, idx: (idx[0], 0, h)),
            pl.BlockSpec((1, c, _H_BLK), lambda h, idx: (idx[1], 0, h)),
            pl.BlockSpec((1, 1, _H_BLK), lambda h, idx: (idx[0], 0, h)),
            pl.BlockSpec((1, 1, _H_BLK), lambda h, idx: (idx[1], 0, h)),
            pl.BlockSpec((1, _H_BLK, c), lambda h, idx: (idx[0], h, 0)),
            pl.BlockSpec((1, _H_BLK, c), lambda h, idx: (idx[1], h, 0)),
            pl.BlockSpec((1, 1, c), lambda h, idx: (idx[0], 0, 0)),
            pl.BlockSpec((1, 1, c), lambda h, idx: (idx[1], 0, 0)),
        ],
        out_specs=pl.BlockSpec((n, c), lambda h, idx: (0, 0)),
    )
    be1r = be1.reshape(e, 1, hid)
    be2r = be2.reshape(e, 1, c)
    out = pl.pallas_call(
        _expert_kernel,
        grid_spec=grid_spec,
        out_shape=jax.ShapeDtypeStruct((n, c), jnp.float32),
    )(idx8.reshape(-1), wvec, xbf,
      We1, We1, be1r, be1r, We2, We2, be2r, be2r)

    output = out.reshape(b, n, c)
    local_ent = stats[0, 0]
    global_ent = stats[0, 1]
    ortho = stats[0, 2]
    expert_mask_mean = emask[0]
    return (output, local_ent, global_ent, ortho, expert_mask_mean,
            probs, topk_i, twf)


# speedup vs baseline: 1.1169x; 1.1169x over previous
import jax
import jax.numpy as jnp
from jax import lax
from jax.experimental import pallas as pl
from jax.experimental.pallas import tpu as pltpu

_PROMPT_K = 16
_TOPK = 2
_H_BLK = 512


def _gelu(v):
    return 0.5 * v * (1.0 + lax.erf(v * 0.7071067811865476))


def _row_select(mat, row):
    e, w = mat.shape
    iota = lax.broadcasted_iota(jnp.int32, (e, w), 0)
    return jnp.sum(jnp.where(iota == row, mat, 0.0), axis=0, keepdims=True)


def _mega_kernel(x_ref, w1_ref, b1_ref, w2_ref, b2_ref, be1_ref, be2_ref,
                 we1_hbm, we2_hbm,
                 probs_ref, stats_ref, emask_ref, twf_ref, idx_ref, out_ref,
                 xbf_scr, we1_buf, we2_buf, sem1, sem2):
    n, c = x_ref.shape
    e = w2_ref.shape[0]
    hid = be1_ref.shape[1]
    ht = hid // _H_BLK
    bf = jnp.bfloat16
    f32 = jnp.float32

    x = x_ref[...]
    xbf_scr[...] = x.astype(bf)

    xm = jnp.sum(x, axis=0, keepdims=True) * (1.0 / n)
    num = jnp.sum(x * xm, axis=1, keepdims=True)
    xn = jnp.sqrt(jnp.sum(x * x, axis=1, keepdims=True))
    mn = jnp.sqrt(jnp.sum(xm * xm))
    eps = 1e-8
    sim = num / (jnp.maximum(xn, eps) * jnp.maximum(mn, eps))

    iota_n = lax.broadcasted_iota(jnp.int32, (n, 1), 0)

    def body(_, carry):
        simw, mask = carry
        m = jnp.max(simw)
        idx = jnp.min(jnp.where(simw == m, iota_n, n))
        hit = iota_n == idx
        mask = jnp.where(hit, 1.0, mask)
        simw = jnp.where(hit, -jnp.inf, simw)
        return simw, mask

    _, mask = lax.fori_loop(0, _PROMPT_K, body,
                            (sim, jnp.zeros((n, 1), f32)))
    context = jnp.sum(x * mask, axis=0, keepdims=True) * (1.0 / _PROMPT_K)

    h1 = _gelu(lax.dot_general(context, w1_ref[...],
                               (((1,), (1,)), ((), ())),
                               preferred_element_type=f32)
               + b1_ref[...])
    logits = lax.dot_general(h1, w2_ref[...],
                             (((1,), (1,)), ((), ())),
                             preferred_element_type=f32) + b2_ref[...]
    ex = jnp.exp(logits - jnp.max(logits))
    p = ex / jnp.sum(ex)

    iota_e = lax.broadcasted_iota(jnp.int32, (1, e), 1)
    m1 = jnp.max(p)
    i1 = jnp.min(jnp.where(p == m1, iota_e, e))
    p2 = jnp.where(iota_e == i1, -1.0, p)
    m2 = jnp.max(p2)
    i2 = jnp.min(jnp.where(p2 == m2, iota_e, e))
    s = m1 + m2 + 1e-9
    tw0 = m1 / s
    tw1 = m2 / s

    local_ent = jnp.sum(-p * jnp.log(p + 1e-6))
    global_ent = jnp.sum(p * jnp.log(p + 1e-6))

    w2m = w2_ref[...]
    rn = jnp.sqrt(jnp.sum(w2m * w2m, axis=1, keepdims=True))
    wn = w2m / jnp.maximum(rn, 1e-12)
    simm = lax.dot_general(wn, wn, (((1,), (1,)), ((), ())),
                           preferred_element_type=f32)
    eye = (lax.broadcasted_iota(jnp.int32, (e, e), 0)
           == lax.broadcasted_iota(jnp.int32, (e, e), 1)).astype(f32)
    ortho = jnp.sqrt(jnp.sum((simm - eye) ** 2))

    hot1 = iota_e == i1
    hot2 = iota_e == i2
    probs_ref[...] = p
    emask_ref[...] = (hot1 | hot2).astype(f32)
    twf_ref[...] = jnp.where(hot1, tw0, 0.0) + jnp.where(hot2, tw1, 0.0)
    stats_ref[...] = (jnp.where(iota_e == 0, local_ent, 0.0)
                      + jnp.where(iota_e == 1, global_ent, 0.0)
                      + jnp.where(iota_e == 2, ortho, 0.0))
    idx_ref[...] = (jnp.where(iota_e == 0, i1, 0)
                    + jnp.where(iota_e == 1, i2, 0)).astype(jnp.int32)

    be1a = _row_select(be1_ref[...], i1)
    be1b = _row_select(be1_ref[...], i2)
    be2a = _row_select(be2_ref[...], i1)
    be2b = _row_select(be2_ref[...], i2)

    def w1copy(sl, j, ei, h):
        return pltpu.make_async_copy(
            we1_hbm.at[ei, :, pl.ds(h * _H_BLK, _H_BLK)],
            we1_buf.at[sl, j], sem1.at[sl, j])

    def w2copy(sl, j, ei, h):
        return pltpu.make_async_copy(
            we2_hbm.at[ei, pl.ds(h * _H_BLK, _H_BLK), :],
            we2_buf.at[sl, j], sem2.at[sl, j])

    def start(h):
        sl = h % 2
        for j, ei in ((0, i1), (1, i2)):
            w1copy(sl, j, ei, h).start()
            w2copy(sl, j, ei, h).start()

    def wait(h):
        sl = h % 2
        for j, ei in ((0, i1), (1, i2)):
            w1copy(sl, j, ei, h).wait()
            w2copy(sl, j, ei, h).wait()

    start(0)
    for h in range(ht):
        if h + 1 < ht:
            start(h + 1)
        wait(h)
        sl = h % 2
        lo, hi = h * _H_BLK, (h + 1) * _H_BLK
        xb = xbf_scr[...]
        hpa = jnp.dot(xb, we1_buf[sl, 0].astype(bf),
                      preferred_element_type=f32)
        ga = _gelu((hpa + be1a[:, lo:hi]).astype(bf))
        hpb = jnp.dot(xb, we1_buf[sl, 1].astype(bf),
                      preferred_element_type=f32)
        gb = _gelu((hpb + be1b[:, lo:hi]).astype(bf))
        o = (tw0 * jnp.dot(ga, we2_buf[sl, 0].astype(bf),
                           preferred_element_type=f32)
             + tw1 * jnp.dot(gb, we2_buf[sl, 1].astype(bf),
                             preferred_element_type=f32))
        if h == 0:
            out_ref[...] = (tw0 * be2a + tw1 * be2b) + o
        else:
            out_ref[...] += o


def kernel(x, W1, b1, W2, b2, We1, be1, We2, be2):
    b, n, c = x.shape
    e, _, hid = We1.shape
    xs = x.reshape(n, c)

    probs, stats, emask, twf, idx8, out = pl.pallas_call(
        _mega_kernel,
        in_specs=[
            pl.BlockSpec((n, c), lambda: (0, 0)),
            pl.BlockSpec(W1.shape, lambda: (0, 0)),
            pl.BlockSpec((1, b1.shape[0]), lambda: (0, 0)),
            pl.BlockSpec(W2.shape, lambda: (0, 0)),
            pl.BlockSpec((1, b2.shape[0]), lambda: (0, 0)),
            pl.BlockSpec(be1.shape, lambda: (0, 0)),
            pl.BlockSpec(be2.shape, lambda: (0, 0)),
            pl.BlockSpec(memory_space=pltpu.MemorySpace.HBM),
            pl.BlockSpec(memory_space=pltpu.MemorySpace.HBM),
        ],
        out_shape=(
            jax.ShapeDtypeStruct((1, e), jnp.float32),
            jax.ShapeDtypeStruct((1, e), jnp.float32),
            jax.ShapeDtypeStruct((1, e), jnp.float32),
            jax.ShapeDtypeStruct((1, e), jnp.float32),
            jax.ShapeDtypeStruct((1, e), jnp.int32),
            jax.ShapeDtypeStruct((n, c), jnp.float32),
        ),
        scratch_shapes=[
            pltpu.VMEM((n, c), jnp.bfloat16),
            pltpu.VMEM((2, 2, c, _H_BLK), jnp.float32),
            pltpu.VMEM((2, 2, _H_BLK, c), jnp.float32),
            pltpu.SemaphoreType.DMA((2, 2)),
            pltpu.SemaphoreType.DMA((2, 2)),
        ],
    )(xs, W1, b1.reshape(1, -1), W2, b2.reshape(1, -1), be1, be2, We1, We2)

    output = out.reshape(b, n, c)
    local_ent = stats[0, 0]
    global_ent = stats[0, 1]
    ortho = stats[0, 2]
    expert_mask_mean = emask[0]
    topk_i = idx8[:, :_TOPK]
    return (output, local_ent, global_ent, ortho, expert_mask_mean,
            probs, topk_i, twf)


# speedup vs baseline: 1.2426x; 1.1126x over previous
import jax
import jax.numpy as jnp
from jax import lax
from jax.experimental import pallas as pl
from jax.experimental.pallas import tpu as pltpu

_PROMPT_K = 16
_TOPK = 2
_H_BLK = 512


def _gelu(v):
    return 0.5 * v * (1.0 + lax.erf(v * 0.7071067811865476))


def _row_select(mat, row):
    e, w = mat.shape
    iota = lax.broadcasted_iota(jnp.int32, (e, w), 0)
    return jnp.sum(jnp.where(iota == row, mat, 0.0), axis=0, keepdims=True)


def _mega_kernel(x_ref, w1_ref, b1_ref, w2_ref, b2_ref, be1_ref, be2_ref,
                 we1_hbm, we2_hbm,
                 probs_ref, stats_ref, emask_ref, twf_ref, idx_ref, out_ref,
                 xbf_scr, we1_buf, we2_buf, sem1, sem2):
    n, c = x_ref.shape
    e = w2_ref.shape[0]
    hid = be1_ref.shape[1]
    ht = hid // _H_BLK
    bf = jnp.bfloat16
    f32 = jnp.float32

    x = x_ref[...]
    xbf_scr[...] = x.astype(bf)

    ones_n = jnp.ones((n, 1), f32)
    ones_c = jnp.ones((1, c), f32)
    cdot = (((1,), (1,)), ((), ()))
    rdot = (((0,), (0,)), ((), ()))
    xm = lax.dot_general(ones_n, x, rdot,
                         preferred_element_type=f32) * (1.0 / n)
    num = lax.dot_general(xm, x, cdot, preferred_element_type=f32)
    xn = jnp.sqrt(lax.dot_general(ones_c, x * x, cdot,
                                  preferred_element_type=f32))
    mn = jnp.sqrt(jnp.sum(xm * xm))
    eps = 1e-8
    sim = num / (jnp.maximum(xn, eps) * jnp.maximum(mn, eps))

    iota_n = lax.broadcasted_iota(jnp.int32, (1, n), 1)

    def body(_, carry):
        simw, mask = carry
        m = jnp.max(simw)
        idx = jnp.min(jnp.where(simw == m, iota_n, n))
        hit = iota_n == idx
        mask = jnp.where(hit, 1.0, mask)
        simw = jnp.where(hit, -jnp.inf, simw)
        return simw, mask

    _, mask = lax.fori_loop(0, _PROMPT_K, body,
                            (sim, jnp.zeros((1, n), f32)))
    context = lax.dot_general(mask, x, (((1,), (0,)), ((), ())),
                              preferred_element_type=f32) * (1.0 / _PROMPT_K)

    h1 = _gelu(lax.dot_general(context, w1_ref[...],
                               (((1,), (1,)), ((), ())),
                               preferred_element_type=f32)
               + b1_ref[...])
    logits = lax.dot_general(h1, w2_ref[...],
                             (((1,), (1,)), ((), ())),
                             preferred_element_type=f32) + b2_ref[...]
    ex = jnp.exp(logits - jnp.max(logits))
    p = ex / jnp.sum(ex)

    iota_e = lax.broadcasted_iota(jnp.int32, (1, e), 1)
    m1 = jnp.max(p)
    i1 = jnp.min(jnp.where(p == m1, iota_e, e))
    p2 = jnp.where(iota_e == i1, -1.0, p)
    m2 = jnp.max(p2)
    i2 = jnp.min(jnp.where(p2 == m2, iota_e, e))
    s = m1 + m2 + 1e-9
    tw0 = m1 / s
    tw1 = m2 / s

    local_ent = jnp.sum(-p * jnp.log(p + 1e-6))
    global_ent = jnp.sum(p * jnp.log(p + 1e-6))

    w2m = w2_ref[...]
    rn = jnp.sqrt(jnp.sum(w2m * w2m, axis=1, keepdims=True))
    wn = w2m / jnp.maximum(rn, 1e-12)
    simm = lax.dot_general(wn, wn, (((1,), (1,)), ((), ())),
                           preferred_element_type=f32)
    eye = (lax.broadcasted_iota(jnp.int32, (e, e), 0)
           == lax.broadcasted_iota(jnp.int32, (e, e), 1)).astype(f32)
    ortho = jnp.sqrt(jnp.sum((simm - eye) ** 2))

    hot1 = iota_e == i1
    hot2 = iota_e == i2
    probs_ref[...] = p
    emask_ref[...] = (hot1 | hot2).astype(f32)
    twf_ref[...] = jnp.where(hot1, tw0, 0.0) + jnp.where(hot2, tw1, 0.0)
    stats_ref[...] = (jnp.where(iota_e == 0, local_ent, 0.0)
                      + jnp.where(iota_e == 1, global_ent, 0.0)
                      + jnp.where(iota_e == 2, ortho, 0.0))
    idx_ref[...] = (jnp.where(iota_e == 0, i1, 0)
                    + jnp.where(iota_e == 1, i2, 0)).astype(jnp.int32)

    be1a = _row_select(be1_ref[...], i1)
    be1b = _row_select(be1_ref[...], i2)
    be2a = _row_select(be2_ref[...], i1)
    be2b = _row_select(be2_ref[...], i2)

    tiles = (128, 256, 512, 512, 512, 512, 512, 128)
    if sum(tiles) != hid:
        tiles = (_H_BLK,) * (hid // _H_BLK)
    offs = []
    o_ = 0
    for t in tiles:
        offs.append(o_)
        o_ += t
    ht_n = len(tiles)

    def w1copy(sl, j, ei, h):
        return pltpu.make_async_copy(
            we1_hbm.at[ei, :, pl.ds(offs[h], tiles[h])],
            we1_buf.at[sl, j, :, pl.ds(0, tiles[h])], sem1.at[sl, j])

    def w2copy(sl, j, ei, h):
        return pltpu.make_async_copy(
            we2_hbm.at[ei, pl.ds(offs[h], tiles[h]), :],
            we2_buf.at[sl, j, pl.ds(0, tiles[h]), :], sem2.at[sl, j])

    def start(h):
        sl = h % 2
        for j, ei in ((0, i1), (1, i2)):
            w1copy(sl, j, ei, h).start()
            w2copy(sl, j, ei, h).start()

    def wait(h):
        sl = h % 2
        for j, ei in ((0, i1), (1, i2)):
            w1copy(sl, j, ei, h).wait()
            w2copy(sl, j, ei, h).wait()

    start(0)
    for h in range(ht_n):
        if h + 1 < ht_n:
            start(h + 1)
        wait(h)
        sl = h % 2
        lo, hi = offs[h], offs[h] + tiles[h]
        xb = xbf_scr[...]
        hpa = jnp.dot(xb, we1_buf[sl, 0, :, :tiles[h]].astype(bf),
                      preferred_element_type=f32)
        ga = _gelu((hpa + be1a[:, lo:hi]).astype(bf))
        hpb = jnp.dot(xb, we1_buf[sl, 1, :, :tiles[h]].astype(bf),
                      preferred_element_type=f32)
        gb = _gelu((hpb + be1b[:, lo:hi]).astype(bf))
        o = (tw0 * jnp.dot(ga, we2_buf[sl, 0, :tiles[h], :].astype(bf),
                           preferred_element_type=f32)
             + tw1 * jnp.dot(gb, we2_buf[sl, 1, :tiles[h], :].astype(bf),
                             preferred_element_type=f32))
        if h == 0:
            out_ref[...] = (tw0 * be2a + tw1 * be2b) + o
        else:
            out_ref[...] += o


def kernel(x, W1, b1, W2, b2, We1, be1, We2, be2):
    b, n, c = x.shape
    e, _, hid = We1.shape
    xs = x.reshape(n, c)

    probs, stats, emask, twf, idx8, out = pl.pallas_call(
        _mega_kernel,
        in_specs=[
            pl.BlockSpec((n, c), lambda: (0, 0)),
            pl.BlockSpec(W1.shape, lambda: (0, 0)),
            pl.BlockSpec((1, b1.shape[0]), lambda: (0, 0)),
            pl.BlockSpec(W2.shape, lambda: (0, 0)),
            pl.BlockSpec((1, b2.shape[0]), lambda: (0, 0)),
            pl.BlockSpec(be1.shape, lambda: (0, 0)),
            pl.BlockSpec(be2.shape, lambda: (0, 0)),
            pl.BlockSpec(memory_space=pltpu.MemorySpace.HBM),
            pl.BlockSpec(memory_space=pltpu.MemorySpace.HBM),
        ],
        out_shape=(
            jax.ShapeDtypeStruct((1, e), jnp.float32),
            jax.ShapeDtypeStruct((1, e), jnp.float32),
            jax.ShapeDtypeStruct((1, e), jnp.float32),
            jax.ShapeDtypeStruct((1, e), jnp.float32),
            jax.ShapeDtypeStruct((1, e), jnp.int32),
            jax.ShapeDtypeStruct((n, c), jnp.float32),
        ),
        scratch_shapes=[
            pltpu.VMEM((n, c), jnp.bfloat16),
            pltpu.VMEM((2, 2, c, _H_BLK), jnp.float32),
            pltpu.VMEM((2, 2, _H_BLK, c), jnp.float32),
            pltpu.SemaphoreType.DMA((2, 2)),
            pltpu.SemaphoreType.DMA((2, 2)),
        ],
    )(xs, W1, b1.reshape(1, -1), W2, b2.reshape(1, -1), be1, be2, We1, We2)

    output = out.reshape(b, n, c)
    local_ent = stats[0, 0]
    global_ent = stats[0, 1]
    ortho = stats[0, 2]
    expert_mask_mean = emask[0]
    topk_i = idx8[:, :_TOPK]
    return (output, local_ent, global_ent, ortho, expert_mask_mean,
            probs, topk_i, twf)
